# Initial kernel scaffold; baseline (speedup 1.0000x reference)
import jax
import jax.numpy as jnp
from jax.experimental import pallas as pl


def kernel(x_vuln, x_patch, ei_vuln_AST, ei_vuln_DDG, ei_vuln_CFG, ei_patch_AST, ei_patch_DDG, ei_patch_CFG, proj_W_vuln, proj_b_vuln, proj_W_patch, proj_b_patch, gat_W, gat_a_src, gat_a_dst, gat_b, bn_gamma, bn_beta, graph_proj_W, graph_proj_b, cls_W, cls_b):
    raise NotImplementedError("write your pallas kernel here")


# speedup vs baseline: 1.0418x; 1.0418x over previous
import jax
import jax.numpy as jnp
from jax.experimental import pallas as pl
from jax.experimental.pallas import tpu as pltpu

N = 10000
E = 160000
D = 256
HID = 256
H = 8
C = 32
L = 3


def _head_kernel(g_ref, w1_ref, b1_ref, w2_ref, b2_ref, o_ref):
    g = jax.nn.relu(g_ref[...] @ w1_ref[...] + b1_ref[...])
    o_ref[...] = jax.nn.sigmoid(g @ w2_ref[...] + b2_ref[...])


def _head(g, w1, b1, w2, b2):
    return pl.pallas_call(
        _head_kernel,
        out_shape=jax.ShapeDtypeStruct((1, 1), jnp.float32),
    )(g, w1, b1[None, :], w2, b2[None, :])


def _gat(x, ei, W, a_s, a_d, b):
    n = x.shape[0]
    h = (x @ W).reshape(n, H, C)
    al_s = jnp.sum(h * a_s, axis=-1)
    al_d = jnp.sum(h * a_d, axis=-1)
    src, dst = ei[0], ei[1]
    e = jax.nn.leaky_relu(al_s[src] + al_d[dst], negative_slope=0.2)
    w = jnp.exp(e)
    den = jax.ops.segment_sum(w, dst, num_segments=n)
    U = jax.ops.segment_sum(h[src] * w[:, :, None], dst, num_segments=n)
    out = U / (den[:, :, None] + 1e-16)
    return out.reshape(n, H * C) + b


def kernel(x_vuln, x_patch, ei_vuln_AST, ei_vuln_DDG, ei_vuln_CFG, ei_patch_AST, ei_patch_DDG, ei_patch_CFG, proj_W_vuln, proj_b_vuln, proj_W_patch, proj_b_patch, gat_W, gat_a_src, gat_a_dst, gat_b, bn_gamma, bn_beta, graph_proj_W, graph_proj_b, cls_W, cls_b):
    xs = {0: x_vuln @ proj_W_vuln + proj_b_vuln, 1: x_patch @ proj_W_patch + proj_b_patch}
    eis = {0: [ei_vuln_AST, ei_vuln_DDG, ei_vuln_CFG], 1: [ei_patch_AST, ei_patch_DDG, ei_patch_CFG]}
    inv_bn_std = 1.0 / jnp.sqrt(1.0 + 1e-5)
    for i in range(L):
        new = {}
        for t in (0, 1):
            acc = 0.0
            for e in range(3):
                acc = acc + _gat(xs[t], eis[t][e], gat_W[i, t, e], gat_a_src[i, t, e], gat_a_dst[i, t, e], gat_b[i, t, e])
            h = acc / 3.0
            h = h * inv_bn_std * bn_gamma[i, t] + bn_beta[i, t]
            h = jax.nn.relu(h)
            new[t] = h + xs[t]
        xs = new
    pools = []
    for t in (0, 1):
        pools.append(jnp.concatenate([jnp.mean(xs[t], axis=0, keepdims=True), jnp.max(xs[t], axis=0, keepdims=True)], axis=1))
    g = jnp.concatenate(pools, axis=1)
    return _head(g, graph_proj_W, graph_proj_b, cls_W, cls_b)


# speedup vs baseline: 43.9106x; 42.1504x over previous
import functools

import jax
import jax.numpy as jnp
from jax import lax
from jax.experimental import pallas as pl
from jax.experimental.pallas import tpu as pltpu
from jax.experimental.pallas import tpu_sc as plsc

N = 10000
E = 160000
D = 256
HID = 256
H = 8
C = 32
L = 3

NC = 2
NS = 16
LANES = 16

CW = 128
NCH = 1280
E_PAD = NCH * CW
NREAL = E // CW
CPC = NCH // NC
CPT = CPC // NS
NROW = 10016
ZR = 624
WR = 624


def _proj_kernel(x_ref, w_ref, b_ref, o_ref):
    o_ref[...] = jnp.dot(x_ref[...], w_ref[...],
                         preferred_element_type=jnp.float32) + b_ref[...]


def _proj(x, w, b):
    return pl.pallas_call(
        _proj_kernel,
        grid=(10,),
        in_specs=[
            pl.BlockSpec((1000, D), lambda r: (r, 0)),
            pl.BlockSpec((D, HID), lambda r: (0, 0)),
            pl.BlockSpec((1, HID), lambda r: (0, 0)),
        ],
        out_specs=pl.BlockSpec((1000, HID), lambda r: (r, 0)),
        out_shape=jax.ShapeDtypeStruct((N, HID), jnp.float32),
    )(x, w, b[None, :])


def _prep_kernel(x_ref, w_ref, as_ref, ad_ref, lo_ref, hi_ref, als_ref, ald_ref):
    h = jnp.dot(x_ref[...], w_ref[0], preferred_element_type=jnp.float32)
    lo_ref[0] = h[:, :128]
    hi_ref[0] = h[:, 128:]
    als_ref[0] = jnp.dot(h, as_ref[0], preferred_element_type=jnp.float32)
    ald_ref[0] = jnp.dot(h, ad_ref[0], preferred_element_type=jnp.float32)


def _prep(x, w3, as3, ad3):
    return pl.pallas_call(
        _prep_kernel,
        grid=(3, 10),
        in_specs=[
            pl.BlockSpec((1000, HID), lambda e, r: (r, 0)),
            pl.BlockSpec((1, HID, HID), lambda e, r: (e, 0, 0)),
            pl.BlockSpec((1, HID, 16), lambda e, r: (e, 0, 0)),
            pl.BlockSpec((1, HID, 16), lambda e, r: (e, 0, 0)),
        ],
        out_specs=[
            pl.BlockSpec((1, 1000, 128), lambda e, r: (e, r, 0)),
            pl.BlockSpec((1, 1000, 128), lambda e, r: (e, r, 0)),
            pl.BlockSpec((1, 1000, 16), lambda e, r: (e, r, 0)),
            pl.BlockSpec((1, 1000, 16), lambda e, r: (e, r, 0)),
        ],
        out_shape=[
            jax.ShapeDtypeStruct((3, N, 128), jnp.float32),
            jax.ShapeDtypeStruct((3, N, 128), jnp.float32),
            jax.ShapeDtypeStruct((3, N, 16), jnp.float32),
            jax.ShapeDtypeStruct((3, N, 16), jnp.float32),
        ],
    )(x, w3, as3, ad3)


def _bn_kernel(u_ref, den_ref, erep_ref, g_ref, c_ref, x_ref, o_ref):
    acc = jnp.zeros((1000, HID), jnp.float32)
    for e in range(3):
        den = den_ref[e, 0] + den_ref[e, 1]
        recip = 1.0 / (den + 1e-16)
        rexp = jnp.dot(recip, erep_ref[...], preferred_element_type=jnp.float32)
        u = jnp.concatenate(
            [u_ref[e, 0, 0] + u_ref[e, 1, 0], u_ref[e, 0, 1] + u_ref[e, 1, 1]],
            axis=1)
        acc = acc + u * rexp
    h = jnp.maximum(acc * g_ref[...] + c_ref[...], 0.0)
    o_ref[...] = h + x_ref[...]


def _bn_residual(u, den, erep, gvec, cvec, x):
    return pl.pallas_call(
        _bn_kernel,
        grid=(10,),
        in_specs=[
            pl.BlockSpec((3, 2, 2, 1000, 128), lambda r: (0, 0, 0, r, 0)),
            pl.BlockSpec((3, 2, 1000, 16), lambda r: (0, 0, r, 0)),
            pl.BlockSpec((16, HID), lambda r: (0, 0)),
            pl.BlockSpec((1, HID), lambda r: (0, 0)),
            pl.BlockSpec((1, HID), lambda r: (0, 0)),
            pl.BlockSpec((1000, HID), lambda r: (r, 0)),
        ],
        out_specs=pl.BlockSpec((1000, HID), lambda r: (r, 0)),
        out_shape=jax.ShapeDtypeStruct((N, HID), jnp.float32),
    )(u, den, erep, gvec[None, :], cvec[None, :], x)


def _head_kernel(x0_ref, x1_ref, w1_ref, b1_ref, w2_ref, b2_ref, o_ref):
    feats = []
    for xr in (x0_ref, x1_ref):
        xv = xr[...]
        feats.append(jnp.mean(xv, axis=0, keepdims=True))
        feats.append(jnp.max(xv, axis=0, keepdims=True))
    g = jnp.concatenate([feats[0], feats[1], feats[2], feats[3]], axis=1)
    g = jnp.maximum(jnp.dot(g, w1_ref[...], preferred_element_type=jnp.float32)
                    + b1_ref[...], 0.0)
    o_ref[...] = jax.nn.sigmoid(
        jnp.dot(g, w2_ref[...], preferred_element_type=jnp.float32) + b2_ref[...])


def _head(x0, x1, w1, b1, w2, b2):
    return pl.pallas_call(
        _head_kernel,
        out_shape=jax.ShapeDtypeStruct((1, 1), jnp.float32),
    )(x0, x1, w1, b1[None, :], w2, b2[None, :])


def _sc_edge_kernel(src_hbm, dst_hbm, als_hbm, ald_hbm, hlo_hbm, hhi_hbm,
                    u_out, den_out,
                    srcbuf, dstbuf, alsrows, aldrows, hrows, wchunk,
                    zbuf, dzbuf, ush, densh,
                    sem_a, sem_b, sem_h):
    cid = lax.axis_index("c")
    sid = lax.axis_index("s")
    base = cid * CPC + sid * CPT
    nreal = jnp.clip(NREAL - base, 0, CPT)

    zero16 = jnp.zeros((LANES,), jnp.float32)

    def _zero_row(r, _):
        for v in range(128 // LANES):
            zbuf[r, pl.ds(v * LANES, LANES)] = zero16
        dzbuf[r, :] = zero16
        return 0

    lax.fori_loop(0, 16, _zero_row, 0)

    def _zero_tables():
        row0 = sid * ZR

        def _z(k, _):
            pltpu.sync_copy(zbuf, ush.at[pl.ds(row0 + k * 16, 16)])
            pltpu.sync_copy(dzbuf, densh.at[pl.ds(row0 + k * 16, 16)])
            return 0

        nz = jnp.where(sid == NS - 1, (ZR + 32) // 16, ZR // 16)
        lax.fori_loop(0, nz, _z, 0)

    _zero_tables()
    plsc.subcore_barrier()

    pltpu.sync_copy(src_hbm.at[pl.ds(base, CPT)], srcbuf)
    pltpu.sync_copy(dst_hbm.at[pl.ds(base, CPT)], dstbuf)

    for half in (0, 1):
        h_hbm = hlo_hbm if half == 0 else hhi_hbm

        def _chunk(j, _):
            cp_a = pltpu.async_copy(als_hbm.at[srcbuf.at[j]], alsrows, sem_a)
            cp_b = pltpu.async_copy(ald_hbm.at[dstbuf.at[j]], aldrows, sem_b)
            cp_h = pltpu.async_copy(h_hbm.at[srcbuf.at[j]], hrows, sem_h)
            cp_a.wait()
            cp_b.wait()

            cp_h.wait()

            def _edge(e, _):
                logit = alsrows[e, :] + aldrows[e, :]
                lr = jnp.where(logit >= 0.0, logit, 0.2 * logit)
                w16 = jnp.exp(lr)
                if half == 0:
                    wchunk[e, :] = w16
                for hd in range(4):
                    wb = jnp.full((LANES,), w16[4 * half + hd], jnp.float32)
                    for v in (2 * hd, 2 * hd + 1):
                        hv = hrows[e, pl.ds(v * LANES, LANES)]
                        hrows[e, pl.ds(v * LANES, LANES)] = hv * wb
                return 0

            lax.fori_loop(0, CW, _edge, 0)

            if half == 0:
                pltpu.sync_copy(wchunk, densh.at[dstbuf.at[j]], add=True)
            pltpu.sync_copy(hrows, ush.at[dstbuf.at[j]], add=True)
            return 0

        lax.fori_loop(0, nreal, _chunk, 0)
        plsc.subcore_barrier()

        row0 = sid * WR
        pltpu.sync_copy(ush.at[pl.ds(row0, WR)],
                        u_out.at[cid, half, pl.ds(row0, WR)])
        if half == 0:
            pltpu.sync_copy(densh.at[pl.ds(row0, WR)],
                            den_out.at[cid, pl.ds(row0, WR)])

        @pl.when(sid == NS - 1)
        def _tail_wb():
            pltpu.sync_copy(ush.at[pl.ds(NS * WR, N - NS * WR)],
                            u_out.at[cid, half, pl.ds(NS * WR, N - NS * WR)])
            if half == 0:
                pltpu.sync_copy(densh.at[pl.ds(NS * WR, N - NS * WR)],
                                den_out.at[cid, pl.ds(NS * WR, N - NS * WR)])

        if half == 0:
            _zero_tables()
        plsc.subcore_barrier()


@functools.partial(jax.jit, static_argnums=())
def _sc_edge(src2d, dst2d, als, ald, hlo, hhi):
    mesh = plsc.VectorSubcoreMesh(core_axis_name="c", subcore_axis_name="s",
                                  num_cores=NC, num_subcores=NS)
    f = pl.kernel(
        _sc_edge_kernel,
        compiler_params=pltpu.CompilerParams(use_tc_tiling_on_sc=False),
        out_type=[
            jax.ShapeDtypeStruct((NC, 2, N, 128), jnp.float32),
            jax.ShapeDtypeStruct((NC, N, 16), jnp.float32),
        ],
        mesh=mesh,
        scratch_types=[
            pltpu.VMEM((CPT, CW), jnp.int32),
            pltpu.VMEM((CPT, CW), jnp.int32),
            pltpu.VMEM((CW, 16), jnp.float32),
            pltpu.VMEM((CW, 16), jnp.float32),
            pltpu.VMEM((CW, 128), jnp.float32),
            pltpu.VMEM((CW, 16), jnp.float32),
            pltpu.VMEM((16, 128), jnp.float32),
            pltpu.VMEM((16, 16), jnp.float32),
            pltpu.VMEM_SHARED((NROW, 128), jnp.float32),
            pltpu.VMEM_SHARED((NROW, 16), jnp.float32),
            pltpu.SemaphoreType.DMA,
            pltpu.SemaphoreType.DMA,
            pltpu.SemaphoreType.DMA,
        ],
    )
    return f(src2d, dst2d, als, ald, hlo, hhi)


def _expand_a(a):
    hid_idx = jnp.arange(HID)
    lane_idx = jnp.arange(16)
    head_of_hid = hid_idx // C
    head_of_lane = lane_idx % H
    mask = (head_of_hid[:, None] == head_of_lane[None, :]).astype(jnp.float32)
    vals = a.reshape(HID)[:, None]
    return mask * vals


def kernel(x_vuln, x_patch, ei_vuln_AST, ei_vuln_DDG, ei_vuln_CFG,
           ei_patch_AST, ei_patch_DDG, ei_patch_CFG,
           proj_W_vuln, proj_b_vuln, proj_W_patch, proj_b_patch,
           gat_W, gat_a_src, gat_a_dst, gat_b, bn_gamma, bn_beta,
           graph_proj_W, graph_proj_b, cls_W, cls_b):
    eis = {0: [ei_vuln_AST, ei_vuln_DDG, ei_vuln_CFG],
           1: [ei_patch_AST, ei_patch_DDG, ei_patch_CFG]}

    pad = E_PAD - E
    edge2d = {}
    for t in (0, 1):
        for e in range(3):
            ei = eis[t][e]
            src = jnp.concatenate([ei[0], jnp.zeros((pad,), jnp.int32)])
            dst = jnp.concatenate([ei[1], jnp.full((pad,), N, jnp.int32)])
            edge2d[(t, e)] = (src.reshape(NCH, CW), dst.reshape(NCH, CW))

    as_m = jnp.stack([jnp.stack([jnp.stack([_expand_a(gat_a_src[i, t, e])
                                            for e in range(3)])
                                 for t in range(2)]) for i in range(L)])
    ad_m = jnp.stack([jnp.stack([jnp.stack([_expand_a(gat_a_dst[i, t, e])
                                            for e in range(3)])
                                 for t in range(2)]) for i in range(L)])

    erep = ((jnp.arange(HID)[None, :] // C) == jnp.arange(16)[:, None]
            ).astype(jnp.float32)

    inv_bn_std = 1.0 / jnp.sqrt(1.0 + 1e-5)

    xs = {0: _proj(x_vuln, proj_W_vuln, proj_b_vuln),
          1: _proj(x_patch, proj_W_patch, proj_b_patch)}

    for i in range(L):
        new = {}
        for t in (0, 1):
            hlo, hhi, als, ald = _prep(xs[t], gat_W[i, t], as_m[i, t], ad_m[i, t])
            us, dens = [], []
            for e in range(3):
                src2d, dst2d = edge2d[(t, e)]
                u, den = _sc_edge(src2d, dst2d, als[e], ald[e], hlo[e], hhi[e])
                us.append(u)
                dens.append(den)
            u_all = jnp.stack(us)
            den_all = jnp.stack(dens)
            gvec = inv_bn_std * bn_gamma[i, t] / 3.0
            bsum = gat_b[i, t, 0] + gat_b[i, t, 1] + gat_b[i, t, 2]
            cvec = bsum * gvec + bn_beta[i, t]
            new[t] = _bn_residual(u_all, den_all, erep, gvec, cvec, xs[t])
        xs = new

    return _head(xs[0], xs[1], graph_proj_W, graph_proj_b, cls_W, cls_b)
